# Initial kernel scaffold; baseline (speedup 1.0000x reference)
import jax
import jax.numpy as jnp
from jax.experimental import pallas as pl


def kernel(inputs, hx, adj_mx, W_ru, b_ru, W_c, b_c):
    raise NotImplementedError("write your pallas kernel here")


# speedup vs baseline: 2.2019x; 2.2019x over previous
import functools

import jax
import jax.numpy as jnp
from jax.experimental import pallas as pl
from jax.experimental.pallas import tpu as pltpu

NUM_NODES = 1024
INPUT_DIM = 2
NUM_UNITS = 64
IN_SZ = INPUT_DIM + NUM_UNITS
NM = 3


def _dcgru_kernel(x0_ref, h_ref, adj_ref, wru_ref, bru_ref, wc_ref, bcb_ref,
                  out_ref, as_ref, x1s, x2s, x0ps, us, *, bc):
    n = NUM_NODES

    @pl.when(pl.program_id(0) == 0)
    def _():
        adj = adj_ref[...]
        d = jnp.sum(adj, axis=1)
        dinv = jnp.where(d > 0.0, 1.0 / d, 0.0)
        as_ref[...] = dinv[:, None] * adj

    a_s = as_ref[...]

    def smat(x):
        return jax.lax.dot_general(
            a_s, x, dimension_numbers=(((0,), (0,)), ((), ())),
            preferred_element_type=jnp.float32)

    x0 = x0_ref[0]
    x1 = smat(x0)
    x2 = 2.0 * smat(x1) - x0
    x1s[...] = x1
    x2s[...] = x2

    for b in range(bc):
        ds = pl.ds(b * IN_SZ, IN_SZ)
        x0b = x0_ref[0, :, ds]
        y = jnp.dot(x0b, wru_ref[0], preferred_element_type=jnp.float32)
        y += jnp.dot(x1s[:, ds], wru_ref[1], preferred_element_type=jnp.float32)
        y += jnp.dot(x2s[:, ds], wru_ref[2], preferred_element_type=jnp.float32)
        ru = jax.nn.sigmoid(y + bru_ref[...])
        r = ru[:, :NUM_UNITS]
        u = ru[:, NUM_UNITS:]
        hb = h_ref[0, :, pl.ds(b * NUM_UNITS, NUM_UNITS)]
        us[:, pl.ds(b * NUM_UNITS, NUM_UNITS)] = u
        x0ps[:, ds] = jnp.concatenate([x0b[:, :INPUT_DIM], r * hb], axis=1)

    x0p = x0ps[...]
    x1p = smat(x0p)
    x2p = 2.0 * smat(x1p) - x0p
    x1s[...] = x1p
    x2s[...] = x2p

    for b in range(bc):
        ds = pl.ds(b * IN_SZ, IN_SZ)
        y = jnp.dot(x0ps[:, ds], wc_ref[0], preferred_element_type=jnp.float32)
        y += jnp.dot(x1s[:, ds], wc_ref[1], preferred_element_type=jnp.float32)
        y += jnp.dot(x2s[:, ds], wc_ref[2], preferred_element_type=jnp.float32)
        c = jnp.tanh(y + bcb_ref[...])
        du = pl.ds(b * NUM_UNITS, NUM_UNITS)
        u = us[:, du]
        hb = h_ref[0, :, du]
        out_ref[0, :, du] = u * hb + (1.0 - u) * c


@jax.jit
def kernel(inputs, hx, adj_mx, W_ru, b_ru, W_c, b_c):
    batch = inputs.shape[0]
    n = NUM_NODES
    bc = 8
    grid = batch // bc

    x_bni = jnp.concatenate(
        [inputs.reshape(batch, n, INPUT_DIM), hx.reshape(batch, n, NUM_UNITS)],
        axis=2)
    x0 = (x_bni.reshape(grid, bc, n, IN_SZ).transpose(0, 2, 1, 3)
          .reshape(grid, n, bc * IN_SZ))
    h_t = (hx.reshape(grid, bc, n, NUM_UNITS).transpose(0, 2, 1, 3)
           .reshape(grid, n, bc * NUM_UNITS))

    wru = W_ru.reshape(IN_SZ, NM, 2 * NUM_UNITS).transpose(1, 0, 2)
    wc = W_c.reshape(IN_SZ, NM, NUM_UNITS).transpose(1, 0, 2)

    out = pl.pallas_call(
        functools.partial(_dcgru_kernel, bc=bc),
        grid=(grid,),
        in_specs=[
            pl.BlockSpec((1, n, bc * IN_SZ), lambda g: (g, 0, 0)),
            pl.BlockSpec((1, n, bc * NUM_UNITS), lambda g: (g, 0, 0)),
            pl.BlockSpec((n, n), lambda g: (0, 0)),
            pl.BlockSpec((NM, IN_SZ, 2 * NUM_UNITS), lambda g: (0, 0, 0)),
            pl.BlockSpec((1, 2 * NUM_UNITS), lambda g: (0, 0)),
            pl.BlockSpec((NM, IN_SZ, NUM_UNITS), lambda g: (0, 0, 0)),
            pl.BlockSpec((1, NUM_UNITS), lambda g: (0, 0)),
        ],
        out_specs=pl.BlockSpec((1, n, bc * NUM_UNITS), lambda g: (g, 0, 0)),
        out_shape=jax.ShapeDtypeStruct((grid, n, bc * NUM_UNITS), jnp.float32),
        scratch_shapes=[
            pltpu.VMEM((n, n), jnp.float32),
            pltpu.VMEM((n, bc * IN_SZ), jnp.float32),
            pltpu.VMEM((n, bc * IN_SZ), jnp.float32),
            pltpu.VMEM((n, bc * IN_SZ), jnp.float32),
            pltpu.VMEM((n, bc * NUM_UNITS), jnp.float32),
        ],
        compiler_params=pltpu.CompilerParams(
            dimension_semantics=("arbitrary",),
        ),
    )(x0, h_t, adj_mx, wru, b_ru[None, :], wc, b_c[None, :])

    return (out.reshape(grid, n, bc, NUM_UNITS).transpose(0, 2, 1, 3)
            .reshape(batch, n * NUM_UNITS))


# speedup vs baseline: 3.3510x; 1.5219x over previous
import functools

import jax
import jax.numpy as jnp
from jax.experimental import pallas as pl
from jax.experimental.pallas import tpu as pltpu

NUM_NODES = 1024
INPUT_DIM = 2
NUM_UNITS = 64
IN_SZ = INPUT_DIM + NUM_UNITS
NM = 3


def _dcgru_kernel(xin_ref, h_ref, adj_ref, wru_ref, bru_ref, wc_ref, bcb_ref,
                  out_ref, as_ref, x0s, x1s, x2s, x0ps, us, *, bc):
    n = NUM_NODES

    @pl.when(pl.program_id(0) == 0)
    def _():
        adj = adj_ref[...]
        d = jnp.sum(adj, axis=1)
        dinv = jnp.where(d > 0.0, 1.0 / d, 0.0)
        as_ref[...] = dinv[:, None] * adj

    a_s = as_ref[...]

    def smat(x):
        return jax.lax.dot_general(
            a_s, x, dimension_numbers=(((0,), (0,)), ((), ())),
            preferred_element_type=jnp.float32)

    for b in range(bc):
        x0s[:, pl.ds(b * IN_SZ, IN_SZ)] = jnp.concatenate(
            [xin_ref[0, :, pl.ds(b * INPUT_DIM, INPUT_DIM)], h_ref[b]], axis=1)

    x0 = x0s[...]
    x1 = smat(x0)
    x2 = 2.0 * smat(x1) - x0
    x1s[...] = x1
    x2s[...] = x2

    for b in range(bc):
        ds = pl.ds(b * IN_SZ, IN_SZ)
        x0b = x0s[:, ds]
        y = jnp.dot(x0b, wru_ref[0], preferred_element_type=jnp.float32)
        y += jnp.dot(x1s[:, ds], wru_ref[1], preferred_element_type=jnp.float32)
        y += jnp.dot(x2s[:, ds], wru_ref[2], preferred_element_type=jnp.float32)
        ru = jax.nn.sigmoid(y + bru_ref[...])
        r = ru[:, :NUM_UNITS]
        u = ru[:, NUM_UNITS:]
        hb = h_ref[b]
        us[:, pl.ds(b * NUM_UNITS, NUM_UNITS)] = u
        x0ps[:, ds] = jnp.concatenate([x0b[:, :INPUT_DIM], r * hb], axis=1)

    x0p = x0ps[...]
    x1p = smat(x0p)
    x2p = 2.0 * smat(x1p) - x0p
    x1s[...] = x1p
    x2s[...] = x2p

    for b in range(bc):
        ds = pl.ds(b * IN_SZ, IN_SZ)
        y = jnp.dot(x0ps[:, ds], wc_ref[0], preferred_element_type=jnp.float32)
        y += jnp.dot(x1s[:, ds], wc_ref[1], preferred_element_type=jnp.float32)
        y += jnp.dot(x2s[:, ds], wc_ref[2], preferred_element_type=jnp.float32)
        c = jnp.tanh(y + bcb_ref[...])
        u = us[:, pl.ds(b * NUM_UNITS, NUM_UNITS)]
        hb = h_ref[b]
        out_ref[b] = u * hb + (1.0 - u) * c


@jax.jit
def kernel(inputs, hx, adj_mx, W_ru, b_ru, W_c, b_c):
    batch = inputs.shape[0]
    n = NUM_NODES
    bc = 8
    grid = batch // bc

    xin = (inputs.reshape(grid, bc, n, INPUT_DIM).transpose(0, 2, 1, 3)
           .reshape(grid, n, bc * INPUT_DIM))
    h3 = hx.reshape(batch, n, NUM_UNITS)

    wru = W_ru.reshape(IN_SZ, NM, 2 * NUM_UNITS).transpose(1, 0, 2)
    wc = W_c.reshape(IN_SZ, NM, NUM_UNITS).transpose(1, 0, 2)

    out = pl.pallas_call(
        functools.partial(_dcgru_kernel, bc=bc),
        grid=(grid,),
        in_specs=[
            pl.BlockSpec((1, n, bc * INPUT_DIM), lambda g: (g, 0, 0)),
            pl.BlockSpec((bc, n, NUM_UNITS), lambda g: (g, 0, 0)),
            pl.BlockSpec((n, n), lambda g: (0, 0)),
            pl.BlockSpec((NM, IN_SZ, 2 * NUM_UNITS), lambda g: (0, 0, 0)),
            pl.BlockSpec((1, 2 * NUM_UNITS), lambda g: (0, 0)),
            pl.BlockSpec((NM, IN_SZ, NUM_UNITS), lambda g: (0, 0, 0)),
            pl.BlockSpec((1, NUM_UNITS), lambda g: (0, 0)),
        ],
        out_specs=pl.BlockSpec((bc, n, NUM_UNITS), lambda g: (g, 0, 0)),
        out_shape=jax.ShapeDtypeStruct((batch, n, NUM_UNITS), jnp.float32),
        scratch_shapes=[
            pltpu.VMEM((n, n), jnp.float32),
            pltpu.VMEM((n, bc * IN_SZ), jnp.float32),
            pltpu.VMEM((n, bc * IN_SZ), jnp.float32),
            pltpu.VMEM((n, bc * IN_SZ), jnp.float32),
            pltpu.VMEM((n, bc * IN_SZ), jnp.float32),
            pltpu.VMEM((n, bc * NUM_UNITS), jnp.float32),
        ],
        compiler_params=pltpu.CompilerParams(
            dimension_semantics=("arbitrary",),
        ),
    )(xin, h3, adj_mx, wru, b_ru[None, :], wc, b_c[None, :])

    return out.reshape(batch, n * NUM_UNITS)
